# Initial kernel scaffold; baseline (speedup 1.0000x reference)
import jax
import jax.numpy as jnp
from jax.experimental import pallas as pl


def kernel(x_user, x_item, edge_index_follows, edge_index_buys, edge_index_rev_buys, W_follows, W_buys, W_rev_buys):
    raise NotImplementedError("write your pallas kernel here")


# speedup vs baseline: 4.5830x; 4.5830x over previous
import functools

import jax
import jax.numpy as jnp
from jax import lax
from jax.experimental import pallas as pl
from jax.experimental.pallas import tpu as pltpu
from jax.experimental.pallas import tpu_sc as plsc

_K = 80
_NC = 2
_NS = 16
_NW = _NC * _NS


def _round_up(x, m):
    return (x + m - 1) // m * m


@functools.lru_cache(maxsize=None)
def _make_sc_aggregate(N, D, E):
    EPT = E // _NW
    CH = EPT // _K
    assert CH * _K == EPT and EPT * _NW == E
    RPT = N // _NS
    assert RPT % _K == 0
    NB = RPT // _K
    f32 = jnp.float32
    mesh = plsc.VectorSubcoreMesh(core_axis_name="c", subcore_axis_name="s")
    out_type = (
        [jax.ShapeDtypeStruct((_NC, N, D), f32)] * 3
        + [jax.ShapeDtypeStruct((_NW * N,), f32)] * 3
    )

    def body(src_f, dst_f, src_b, dst_b, src_rb, dst_rb,
             x_user, x_item, zeros_feat, zeros_deg,
             agg_f, agg_b, agg_rb, deg_f, deg_b, deg_rb,
             acc, idx, rows, deg, sem):
        c = lax.axis_index("c")
        s = lax.axis_index("s")
        wid = c * _NS + s
        r0 = s * RPT
        ones16 = jnp.ones((16,), f32)
        for srcs, dsts, table, agg_out, deg_out in (
                (src_f, dst_f, x_user, agg_f, deg_f),
                (src_b, dst_b, x_user, agg_b, deg_b),
                (src_rb, dst_rb, x_item, agg_rb, deg_rb)):
            pltpu.sync_copy(zeros_feat.at[pl.ds(0, _K)], rows)
            pltpu.sync_copy(zeros_deg, deg)
            for z in range(NB):
                pltpu.sync_copy(rows, acc.at[pl.ds(r0 + z * _K, _K)])
            plsc.subcore_barrier()
            base = wid * EPT

            def chunk(j, carry):
                off = pl.multiple_of(base + j * _K, 8)
                pltpu.sync_copy(srcs.at[pl.ds(off, _K)], idx.at[0])
                pltpu.sync_copy(dsts.at[pl.ds(off, _K)], idx.at[1])
                pltpu.async_copy(table.at[idx.at[0]], rows, sem).wait()
                pltpu.sync_copy(rows, acc.at[idx.at[1]], add=True)
                for g in range(_K // 16):
                    iv = idx[1, pl.ds(g * 16, 16)]
                    plsc.addupdate_scatter(deg, [iv], ones16)
                return carry

            lax.fori_loop(0, CH, chunk, 0)
            plsc.subcore_barrier()
            for z in range(NB):
                pltpu.sync_copy(acc.at[pl.ds(r0 + z * _K, _K)], rows)
                pltpu.sync_copy(rows, agg_out.at[c, pl.ds(r0 + z * _K, _K)])
            pltpu.sync_copy(deg, deg_out.at[pl.ds(wid * N, N)])

    return pl.kernel(
        body,
        out_type=out_type,
        mesh=mesh,
        compiler_params=pltpu.CompilerParams(needs_layout_passes=False),
        scratch_types=[
            pltpu.VMEM_SHARED((N, D), f32),
            pltpu.VMEM((2, _K), jnp.int32),
            pltpu.VMEM((_K, D), f32),
            pltpu.VMEM((N,), f32),
            pltpu.SemaphoreType.DMA,
        ],
    )


@functools.lru_cache(maxsize=None)
def _make_epilogue(N, Np, D, nrel):
    f32 = jnp.float32
    R = 128

    def body(*args):
        o_ref = args[-1]
        ones = jnp.ones((_NW, 1), f32)
        for r in range(nrel):
            a_ref, d_ref, w_ref = args[3 * r], args[3 * r + 1], args[3 * r + 2]
            p = a_ref[0] + a_ref[1]
            d = d_ref[:, 0, 0, :]
            dcol = lax.dot_general(d, ones, (((0,), (0,)), ((), ())),
                                   preferred_element_type=f32)
            dcol = jnp.maximum(dcol, 1.0)
            o_ref[:, r, :] = jnp.dot(p / dcol, w_ref[...],
                                     preferred_element_type=f32)

    in_specs = []
    for _ in range(nrel):
        in_specs += [
            pl.BlockSpec((_NC, R, D), lambda i: (0, i, 0)),
            pl.BlockSpec((_NW, 1, 1, 128), lambda i: (0, i, 0, 0)),
            pl.BlockSpec((D, D), lambda i: (0, 0)),
        ]
    grid = (pl.cdiv(N, R),)
    return pl.pallas_call(
        body,
        grid=grid,
        in_specs=in_specs,
        out_specs=pl.BlockSpec((R, nrel, D), lambda i: (i, 0, 0)),
        out_shape=jax.ShapeDtypeStruct((N, nrel, D), f32),
    )


def kernel(x_user, x_item, edge_index_follows, edge_index_buys,
           edge_index_rev_buys, W_follows, W_buys, W_rev_buys):
    N, D = x_user.shape
    E = edge_index_follows.shape[1]
    i32 = jnp.int32
    src_f, dst_f = (edge_index_follows[0].astype(i32),
                    edge_index_follows[1].astype(i32))
    src_b, dst_b = (edge_index_buys[0].astype(i32),
                    edge_index_buys[1].astype(i32))
    src_rb, dst_rb = (edge_index_rev_buys[0].astype(i32),
                      edge_index_rev_buys[1].astype(i32))
    f32 = jnp.float32
    Np = _round_up(N, _K * _NS)
    zeros_feat = jnp.zeros((Np, D), f32)
    zeros_deg = jnp.zeros((Np,), f32)

    sc = _make_sc_aggregate(Np, D, E)
    agg_f, agg_b, agg_rb, deg_f, deg_b, deg_rb = sc(
        src_f, dst_f, src_b, dst_b, src_rb, dst_rb,
        x_user.astype(f32), x_item.astype(f32), zeros_feat, zeros_deg)
    deg_f, deg_b, deg_rb = (d.reshape(_NW, Np // 128, 1, 128)
                            for d in (deg_f, deg_b, deg_rb))

    out_user = _make_epilogue(N, Np, D, 2)(
        agg_f, deg_f, W_follows, agg_rb, deg_rb, W_rev_buys)
    out_item = _make_epilogue(N, Np, D, 1)(agg_b, deg_b, W_buys)
    return out_user, out_item
